# baseline (speedup 1.0000x reference)
import functools

import jax
import jax.numpy as jnp
from jax import lax
from jax.experimental import pallas as pl
from jax.experimental.pallas import tpu as pltpu
from jax.experimental.pallas import tpu_sc as plsc

_D = 64
_B_L = 16384
_B_Y = 16384 * 50
_NW = 32
_CHUNK = 128
_LW = 4
_YW = 200


def _body(relidx, allyidx, table, out_l, out_y,
          idx_l, idx_y, bga, bgb, bta, btb, isem, gsa, gsb, osem):
    wid = lax.axis_index("s") * 2 + lax.axis_index("c")

    pltpu.sync_copy(relidx.at[pl.ds(wid * _LW, _LW)], idx_l)
    icp = pltpu.async_copy(allyidx.at[pl.ds(wid * _YW, _YW)], idx_y, isem)

    iota = lax.iota(jnp.int32, 16)
    rot = [(iota + s) % 16 for s in range(16)]
    sct = [((iota + s) % 16) * 128 + iota for s in range(16)]

    def transpose(bg, bt):
        for d0 in range(4):
            cols = [rot[s] + d0 * 16 for s in range(16)]

            @plsc.parallel_loop(0, 4)
            def _(b, d0=d0, cols=cols):
                il0a = b * 16
                il0b = (b + 4) * 16
                rows_a = iota + il0a
                rows_b = iota + il0b
                off_a = d0 * 2048 + il0a
                off_b = d0 * 2048 + il0b
                for s in range(16):
                    va = plsc.load_gather(bg, [rows_a, cols[s]])
                    vb = plsc.load_gather(bg, [rows_b, cols[s]])
                    plsc.store_scatter(bt, [sct[s] + off_a], va)
                    plsc.store_scatter(bt, [sct[s] + off_b], vb)

    def fire_out(bt, out, j, c, sem):
        return [
            pltpu.async_copy(
                bt.at[pl.ds(dh * 1024, 1024)],
                out.at[pl.ds(((j * 8 + dh) * 128 + c) * 1024, 1024)], sem)
            for dh in range(8)
        ]

    for r in range(_LW):
        pltpu.async_copy(table.at[idx_l.at[r]], bga, gsa).wait()
        transpose(bga, bta)
        for o in fire_out(bta, out_l, 0, wid * _LW + r, osem):
            o.wait()

    icp.wait()
    pltpu.async_copy(table.at[idx_y.at[0]], bga, gsa)

    @pl.loop(0, _YW // 2)
    def _(t):
        u0 = 2 * t
        uu0 = wid * _YW + u0
        pltpu.async_copy(table.at[idx_y.at[u0 + 1]], bgb, gsb)
        pltpu.make_async_copy(table.at[idx_y.at[u0]], bga, gsa).wait()
        transpose(bga, bta)
        oa = fire_out(bta, out_y, uu0 // 128, uu0 % 128, osem)
        nxt = jnp.minimum(u0 + 2, _YW - 1)
        pltpu.async_copy(table.at[idx_y.at[nxt]], bga, gsa)
        pltpu.make_async_copy(table.at[idx_y.at[u0 + 1]], bgb, gsb).wait()
        transpose(bgb, btb)
        ob = fire_out(btb, out_y, (uu0 + 1) // 128, (uu0 + 1) % 128, osem)
        for o in oa + ob:
            o.wait()

    pltpu.make_async_copy(table.at[idx_y.at[_YW - 1]], bga, gsa).wait()


@functools.partial(jax.jit, donate_argnums=())
def kernel(relation, all_y, relation_emb_weight):
    relidx = relation.reshape(_B_L // _CHUNK, _CHUNK)
    allyidx = all_y.T.reshape(_B_Y // _CHUNK, _CHUNK)
    mesh = plsc.VectorSubcoreMesh(core_axis_name="c", subcore_axis_name="s")
    out_l, out_y = pl.kernel(
        _body,
        out_type=(
            jax.ShapeDtypeStruct((_B_L * _D,), jnp.float32),
            jax.ShapeDtypeStruct((_B_Y * _D,), jnp.float32),
        ),
        mesh=mesh,
        compiler_params=pltpu.CompilerParams(
            use_tc_tiling_on_sc=False, needs_layout_passes=False),
        scratch_types=[
            pltpu.VMEM((_LW, _CHUNK), jnp.int32),
            pltpu.VMEM((_YW, _CHUNK), jnp.int32),
            pltpu.VMEM((_CHUNK, _D), jnp.float32),
            pltpu.VMEM((_CHUNK, _D), jnp.float32),
            pltpu.VMEM((_CHUNK * _D,), jnp.float32),
            pltpu.VMEM((_CHUNK * _D,), jnp.float32),
            pltpu.SemaphoreType.DMA,
            pltpu.SemaphoreType.DMA,
            pltpu.SemaphoreType.DMA,
            pltpu.SemaphoreType.DMA,
        ],
    )(relidx, allyidx, relation_emb_weight)
    w_l = out_l.reshape(1, 8, 128, 8, 128).transpose(2, 4, 0, 1, 3)
    w_y = out_y.reshape(50, 8, 128, 8, 128).transpose(2, 4, 0, 1, 3)
    return (w_l.reshape(_B_L, 1, _D), relation_emb_weight,
            w_y.reshape(_B_L, 50, _D))


# speedup vs baseline: 1.0216x; 1.0216x over previous
import functools

import jax
import jax.numpy as jnp
from jax import lax
from jax.experimental import pallas as pl
from jax.experimental.pallas import tpu as pltpu
from jax.experimental.pallas import tpu_sc as plsc

_D = 64
_B_L = 16384
_B_Y = 16384 * 50
_NW = 32
_CHUNK = 128
_LW = 4
_YW = 200


def _body(relidx, allyidx, table, out_l, out_y,
          idx_l, idx_y, bga, bgb, bta, btb, isem, gsa, gsb, osem):
    wid = lax.axis_index("s") * 2 + lax.axis_index("c")

    pltpu.sync_copy(relidx.at[pl.ds(wid * _LW, _LW)], idx_l)
    icp = pltpu.async_copy(allyidx.at[pl.ds(wid * _YW, _YW)], idx_y, isem)

    iota = lax.iota(jnp.int32, 16)
    rot = [(iota + s) % 16 for s in range(16)]
    sct = [((iota + s) % 16) * 128 + iota for s in range(16)]

    def transpose(bg, bt):
        for d0 in range(4):
            cols = [rot[s] + d0 * 16 for s in range(16)]

            @plsc.parallel_loop(0, 8)
            def _(b, d0=d0, cols=cols):
                il0 = b * 16
                rows = iota + il0
                off = d0 * 2048 + il0
                for s in range(16):
                    v = plsc.load_gather(bg, [rows, cols[s]])
                    plsc.store_scatter(bt, [sct[s] + off], v)

    def fire_out(bt, out, j, c, sem):
        return [
            pltpu.async_copy(
                bt.at[pl.ds(dh * 1024, 1024)],
                out.at[pl.ds(((j * 8 + dh) * 128 + c) * 1024, 1024)], sem)
            for dh in range(8)
        ]

    for r in range(_LW):
        pltpu.async_copy(table.at[idx_l.at[r]], bga, gsa).wait()
        transpose(bga, bta)
        for o in fire_out(bta, out_l, 0, wid * _LW + r, osem):
            o.wait()

    icp.wait()
    pltpu.async_copy(table.at[idx_y.at[0]], bga, gsa)

    @pl.loop(0, _YW // 2)
    def _(t):
        u0 = 2 * t
        uu0 = wid * _YW + u0
        pltpu.async_copy(table.at[idx_y.at[u0 + 1]], bgb, gsb)
        pltpu.make_async_copy(table.at[idx_y.at[u0]], bga, gsa).wait()
        transpose(bga, bta)
        oa = fire_out(bta, out_y, uu0 // 128, uu0 % 128, osem)
        nxt = jnp.minimum(u0 + 2, _YW - 1)
        pltpu.async_copy(table.at[idx_y.at[nxt]], bga, gsa)
        pltpu.make_async_copy(table.at[idx_y.at[u0 + 1]], bgb, gsb).wait()
        transpose(bgb, btb)
        ob = fire_out(btb, out_y, (uu0 + 1) // 128, (uu0 + 1) % 128, osem)
        for o in oa + ob:
            o.wait()

    pltpu.make_async_copy(table.at[idx_y.at[_YW - 1]], bga, gsa).wait()


@functools.partial(jax.jit, donate_argnums=())
def kernel(relation, all_y, relation_emb_weight):
    relidx = relation.reshape(_B_L // _CHUNK, _CHUNK)
    allyidx = all_y.T.reshape(_B_Y // _CHUNK, _CHUNK)
    mesh = plsc.VectorSubcoreMesh(core_axis_name="c", subcore_axis_name="s")
    out_l, out_y = pl.kernel(
        _body,
        out_type=(
            jax.ShapeDtypeStruct((_B_L * _D,), jnp.float32),
            jax.ShapeDtypeStruct((_B_Y * _D,), jnp.float32),
        ),
        mesh=mesh,
        compiler_params=pltpu.CompilerParams(
            use_tc_tiling_on_sc=False, needs_layout_passes=False),
        scratch_types=[
            pltpu.VMEM((_LW, _CHUNK), jnp.int32),
            pltpu.VMEM((_YW, _CHUNK), jnp.int32),
            pltpu.VMEM((_CHUNK, _D), jnp.float32),
            pltpu.VMEM((_CHUNK, _D), jnp.float32),
            pltpu.VMEM((_CHUNK * _D,), jnp.float32),
            pltpu.VMEM((_CHUNK * _D,), jnp.float32),
            pltpu.SemaphoreType.DMA,
            pltpu.SemaphoreType.DMA,
            pltpu.SemaphoreType.DMA,
            pltpu.SemaphoreType.DMA,
        ],
    )(relidx, allyidx, relation_emb_weight)
    w_l = out_l.reshape(1, 8, 128, 8, 128).transpose(2, 4, 0, 1, 3)
    w_y = out_y.reshape(50, 8, 128, 8, 128).transpose(2, 4, 0, 1, 3)
    one = (relation[0, 0] * 0 + 1).astype(jnp.float32)
    table_out = relation_emb_weight * one
    return (w_l.reshape(_B_L, 1, _D), table_out,
            w_y.reshape(_B_L, 50, _D))


# speedup vs baseline: 1.3082x; 1.2805x over previous
import functools

import jax
import jax.numpy as jnp
from jax import lax
from jax.experimental import pallas as pl
from jax.experimental.pallas import tpu as pltpu
from jax.experimental.pallas import tpu_sc as plsc

_D = 64
_B_L = 16384
_B_Y = 16384 * 50
_NW = 32
_CHUNK = 128
_LW = 4
_YW = 200


def _body(relidx, allyidx, table, out_l, out_y,
          idx_l, idx_y, bga, bgb, bta, btb, isem, gsa, gsb, osem):
    wid = lax.axis_index("s") * 2 + lax.axis_index("c")

    pltpu.sync_copy(relidx.at[pl.ds(wid * _LW, _LW)], idx_l)
    icp = pltpu.async_copy(allyidx.at[pl.ds(wid * _YW, _YW)], idx_y, isem)

    iota = lax.iota(jnp.int32, 16)
    rot = [(iota + s) % 16 for s in range(16)]
    sct = [((iota + s) % 16) * 128 + iota for s in range(16)]

    def transpose(bg, bt):
        for d0 in range(4):
            cols = [rot[s] + d0 * 16 for s in range(16)]

            @plsc.parallel_loop(0, 8)
            def _(b, d0=d0, cols=cols):
                il0 = b * 16
                rows = iota + il0
                off = d0 * 2048 + il0
                vs = [plsc.load_gather(bg, [rows, cols[s]])
                      for s in range(16)]
                for s in range(16):
                    plsc.store_scatter(bt, [sct[s] + off], vs[s])

    def fire_out(bt, out, j, c, sem):
        return [
            pltpu.async_copy(
                bt.at[pl.ds(dh * 1024, 1024)],
                out.at[pl.ds(((j * 8 + dh) * 128 + c) * 1024, 1024)], sem)
            for dh in range(8)
        ]

    for r in range(_LW):
        pltpu.async_copy(table.at[idx_l.at[r]], bga, gsa).wait()
        transpose(bga, bta)
        for o in fire_out(bta, out_l, 0, wid * _LW + r, osem):
            o.wait()

    icp.wait()
    pltpu.async_copy(table.at[idx_y.at[0]], bga, gsa)

    @pl.loop(0, _YW // 2)
    def _(t):
        u0 = 2 * t
        uu0 = wid * _YW + u0
        pltpu.async_copy(table.at[idx_y.at[u0 + 1]], bgb, gsb)
        pltpu.make_async_copy(table.at[idx_y.at[u0]], bga, gsa).wait()
        transpose(bga, bta)
        oa = fire_out(bta, out_y, uu0 // 128, uu0 % 128, osem)
        nxt = jnp.minimum(u0 + 2, _YW - 1)
        pltpu.async_copy(table.at[idx_y.at[nxt]], bga, gsa)
        pltpu.make_async_copy(table.at[idx_y.at[u0 + 1]], bgb, gsb).wait()
        transpose(bgb, btb)
        ob = fire_out(btb, out_y, (uu0 + 1) // 128, (uu0 + 1) % 128, osem)
        for o in oa + ob:
            o.wait()

    pltpu.make_async_copy(table.at[idx_y.at[_YW - 1]], bga, gsa).wait()


@functools.partial(jax.jit, donate_argnums=())
def kernel(relation, all_y, relation_emb_weight):
    relidx = relation.reshape(_B_L // _CHUNK, _CHUNK)
    allyidx = all_y.T.reshape(_B_Y // _CHUNK, _CHUNK)
    mesh = plsc.VectorSubcoreMesh(core_axis_name="c", subcore_axis_name="s")
    out_l, out_y = pl.kernel(
        _body,
        out_type=(
            jax.ShapeDtypeStruct((_B_L * _D,), jnp.float32),
            jax.ShapeDtypeStruct((_B_Y * _D,), jnp.float32),
        ),
        mesh=mesh,
        compiler_params=pltpu.CompilerParams(
            use_tc_tiling_on_sc=False, needs_layout_passes=False),
        scratch_types=[
            pltpu.VMEM((_LW, _CHUNK), jnp.int32),
            pltpu.VMEM((_YW, _CHUNK), jnp.int32),
            pltpu.VMEM((_CHUNK, _D), jnp.float32),
            pltpu.VMEM((_CHUNK, _D), jnp.float32),
            pltpu.VMEM((_CHUNK * _D,), jnp.float32),
            pltpu.VMEM((_CHUNK * _D,), jnp.float32),
            pltpu.SemaphoreType.DMA,
            pltpu.SemaphoreType.DMA,
            pltpu.SemaphoreType.DMA,
            pltpu.SemaphoreType.DMA,
        ],
    )(relidx, allyidx, relation_emb_weight)
    w_l = out_l.reshape(1, 8, 128, 8, 128).transpose(2, 4, 0, 1, 3)
    w_y = out_y.reshape(50, 8, 128, 8, 128).transpose(2, 4, 0, 1, 3)
    one = jnp.where(relation[0, 0] < 0, 2.0, 1.0).astype(jnp.float32)
    table_out = relation_emb_weight * one
    return (w_l.reshape(_B_L, 1, _D), table_out,
            w_y.reshape(_B_L, 50, _D))
